# baseline (speedup 1.0000x reference)
import jax
import jax.numpy as jnp
from jax.experimental import pallas as pl

D_MODEL = 64
_T = 4096
_KM = 32
_KH = 40
_K = 88
_DQ = 65535


def _encoder_block(pc_ref, dlo_ref, dhi_ref,
                   rhs_ref, vw_ref, beta_ref, epsc_ref, out_ref):
    bf16 = jnp.bfloat16
    t2 = _T // 2

    pc = pc_ref[0].astype(jnp.int32)
    dlo = dlo_ref[0].astype(jnp.int32)
    dhi = dhi_ref[0].astype(jnp.int32)

    me = pc & 31
    se = (pc >> 5) & 3
    mo = (pc >> 8) & 31
    so = (pc >> 13) & 3
    dqe = (dlo & 255) | ((dhi & 255) << 8)
    dqo = (dlo >> 8) | ((dhi >> 8) << 8)

    rows = jax.lax.broadcasted_iota(jnp.int32, (_KH, t2), 0)
    oh_e = ((rows == me) | (rows == se + _KM)).astype(bf16)
    oh_o = ((rows == mo) | (rows == so + _KM)).astype(bf16)
    dur_rows = jnp.concatenate(
        [dqe.astype(jnp.float32), dqo.astype(jnp.float32),
         jnp.zeros((6, t2), jnp.float32)], axis=0).astype(bf16)
    lhs = jnp.concatenate([dur_rows, oh_e, oh_o], axis=0)

    dn = (((0,), (0,)), ((), ()))
    c = jax.lax.dot_general(lhs, rhs_ref[...], dn,
                            preferred_element_type=jnp.float32)

    sq = (c * c + epsc_ref[0, 0]).astype(bf16)
    var = jnp.dot(sq, vw_ref[...], preferred_element_type=jnp.float32)
    out_ref[...] = c * jax.lax.rsqrt(var) + beta_ref[...]


def kernel(durations, machine_ids, statuses, W_dur, b_dur,
           machine_table, status_table, gamma, beta):
    B, L, _ = durations.shape
    n = B * L
    n2 = n // 2
    nb = n // _T
    t2 = _T // 2
    f32 = jnp.float32
    bf16 = jnp.bfloat16

    fm = machine_ids.reshape(n).astype(jnp.int32)
    fs = statuses.reshape(n).astype(jnp.int32)
    fd = durations.reshape(n)
    code = (fm + fs * 32).astype(f32)
    dq = jnp.round(fd * _DQ).astype(jnp.int32)
    dlo8 = (dq & 255).astype(f32)
    dhi8 = (dq >> 8).astype(f32)
    i = jnp.arange(256)[:, None]
    j = jnp.arange(128)[None, :]
    pmat = ((i == 2 * j) + 256 * (i == 2 * j + 1)).astype(f32)

    def pack(v):
        y = jnp.dot(v.reshape(n // 256, 256), pmat,
                    precision=jax.lax.Precision.HIGHEST)
        return y.reshape(nb, 1, t2)

    pc = pack(code)
    dlo = pack(dlo8)
    dhi = pack(dhi8)

    cmat = (jnp.eye(D_MODEL, dtype=f32)
            - jnp.full((D_MODEL, D_MODEL), 1.0 / D_MODEL, f32)) * gamma
    mtab = jnp.matmul(machine_table, cmat)
    stab = jnp.matmul(status_table + b_dur, cmat)
    wc = jnp.matmul(W_dur, cmat) / _DQ
    half = jnp.zeros((_KH, D_MODEL), f32)
    half = half.at[:mtab.shape[0]].set(mtab)
    half = half.at[_KM:_KM + stab.shape[0]].set(stab)
    z = jnp.zeros_like(half)
    zw = jnp.zeros_like(wc)
    rhs = jnp.concatenate([
        jnp.concatenate([wc, zw], axis=1),
        jnp.concatenate([zw, wc], axis=1),
        jnp.zeros((6, 2 * D_MODEL), f32),
        jnp.concatenate([half, z], axis=1),
        jnp.concatenate([z, half], axis=1),
    ], axis=0).astype(bf16)

    w1 = 1.0 / (D_MODEL * gamma * gamma)
    wcol = jnp.broadcast_to(w1[:, None], (D_MODEL, D_MODEL))
    zz = jnp.zeros((D_MODEL, D_MODEL), f32)
    vw = jnp.concatenate([
        jnp.concatenate([wcol, zz], axis=1),
        jnp.concatenate([zz, wcol], axis=1),
    ], axis=0).astype(bf16)
    epsc = (1e-5 / jnp.sum(w1)).reshape(1, 1)
    beta2 = jnp.concatenate([beta, beta]).reshape(1, 2 * D_MODEL)

    blk = lambda i: (i, 0, 0)
    full = lambda *shape: pl.BlockSpec(shape, lambda i: (0,) * len(shape))

    out = pl.pallas_call(
        _encoder_block,
        grid=(nb,),
        in_specs=[pl.BlockSpec((1, 1, t2), blk)] * 3 + [
            full(_K, 2 * D_MODEL),
            full(2 * D_MODEL, 2 * D_MODEL),
            full(1, 2 * D_MODEL),
            full(1, 1),
        ],
        out_specs=pl.BlockSpec((t2, 2 * D_MODEL), lambda i: (i, 0)),
        out_shape=jax.ShapeDtypeStruct((n2, 2 * D_MODEL), f32),
    )(pc, dlo, dhi, rhs, vw, beta2, epsc)

    return out.reshape(B, L, D_MODEL)


# speedup vs baseline: 1.0344x; 1.0344x over previous
import jax
import jax.numpy as jnp
from jax.experimental import pallas as pl

D_MODEL = 64
_T = 4096
_R = _T // 256
_KM = 32
_KH = 40
_K = 88


def _encoder_block(mid_ref, sid_ref, dur_ref, pe_ref, po_ref,
                   rhs_ref, vw_ref, beta_ref, epsc_ref, out_ref):
    bf16 = jnp.bfloat16
    f32 = jnp.float32
    t2 = _T // 2

    x = jnp.concatenate([
        mid_ref[...].astype(bf16),
        sid_ref[...].astype(bf16),
        dur_ref[...].astype(bf16),
    ], axis=0)
    ye = jnp.dot(x, pe_ref[...], preferred_element_type=f32)
    yo = jnp.dot(x, po_ref[...], preferred_element_type=f32)
    me = ye[0:_R].astype(jnp.int32)
    se = ye[_R:2 * _R].astype(jnp.int32)
    de = ye[2 * _R:3 * _R].astype(bf16)
    mo = yo[0:_R].astype(jnp.int32)
    so = yo[_R:2 * _R].astype(jnp.int32)
    do = yo[2 * _R:3 * _R].astype(bf16)

    rows = jax.lax.broadcasted_iota(jnp.int32, (_KH, 128), 0)
    ohs_e, ohs_o = [], []
    for s in range(_R):
        ohs_e.append(((rows == me[s:s + 1]) | (rows == se[s:s + 1] + _KM)
                      ).astype(bf16))
        ohs_o.append(((rows == mo[s:s + 1]) | (rows == so[s:s + 1] + _KM)
                      ).astype(bf16))
    oh_e = jnp.concatenate(ohs_e, axis=1)
    oh_o = jnp.concatenate(ohs_o, axis=1)
    dur_e = jnp.concatenate([de[s:s + 1] for s in range(_R)], axis=1)
    dur_o = jnp.concatenate([do[s:s + 1] for s in range(_R)], axis=1)
    zero6 = jnp.zeros((6, t2), bf16)
    lhs = jnp.concatenate([dur_e, dur_o, zero6, oh_e, oh_o], axis=0)

    dn = (((0,), (0,)), ((), ()))
    c = jax.lax.dot_general(lhs, rhs_ref[...], dn,
                            preferred_element_type=f32)

    sq = (c * c + epsc_ref[0, 0]).astype(bf16)
    var = jnp.dot(sq, vw_ref[...], preferred_element_type=f32)
    out_ref[...] = c * jax.lax.rsqrt(var) + beta_ref[...]


def kernel(durations, machine_ids, statuses, W_dur, b_dur,
           machine_table, status_table, gamma, beta):
    B, L, _ = durations.shape
    n = B * L
    n2 = n // 2
    nb = n // _T
    t2 = _T // 2
    f32 = jnp.float32
    bf16 = jnp.bfloat16

    mid2 = machine_ids.astype(jnp.int32).reshape(n // 256, 256)
    sid2 = statuses.astype(jnp.int32).reshape(n // 256, 256)
    dur2 = durations.reshape(n // 256, 256)

    i = jnp.arange(256)[:, None]
    j = jnp.arange(128)[None, :]
    pe = (i == 2 * j).astype(bf16)
    po = (i == 2 * j + 1).astype(bf16)

    cmat = (jnp.eye(D_MODEL, dtype=f32)
            - jnp.full((D_MODEL, D_MODEL), 1.0 / D_MODEL, f32)) * gamma
    mtab = jnp.matmul(machine_table, cmat)
    stab = jnp.matmul(status_table + b_dur, cmat)
    wc = jnp.matmul(W_dur, cmat)
    half = jnp.zeros((_KH, D_MODEL), f32)
    half = half.at[:mtab.shape[0]].set(mtab)
    half = half.at[_KM:_KM + stab.shape[0]].set(stab)
    z = jnp.zeros_like(half)
    zw = jnp.zeros_like(wc)
    rhs = jnp.concatenate([
        jnp.concatenate([wc, zw], axis=1),
        jnp.concatenate([zw, wc], axis=1),
        jnp.zeros((6, 2 * D_MODEL), f32),
        jnp.concatenate([half, z], axis=1),
        jnp.concatenate([z, half], axis=1),
    ], axis=0).astype(bf16)

    w1 = 1.0 / (D_MODEL * gamma * gamma)
    wcol = jnp.broadcast_to(w1[:, None], (D_MODEL, D_MODEL))
    zz = jnp.zeros((D_MODEL, D_MODEL), f32)
    vw = jnp.concatenate([
        jnp.concatenate([wcol, zz], axis=1),
        jnp.concatenate([zz, wcol], axis=1),
    ], axis=0).astype(bf16)
    epsc = (1e-5 / jnp.sum(w1)).reshape(1, 1)
    beta2 = jnp.concatenate([beta, beta]).reshape(1, 2 * D_MODEL)

    blk = lambda i: (i, 0)
    full = lambda *shape: pl.BlockSpec(shape, lambda i: (0,) * len(shape))

    out = pl.pallas_call(
        _encoder_block,
        grid=(nb,),
        in_specs=[pl.BlockSpec((_R, 256), blk)] * 3 + [
            full(256, 128),
            full(256, 128),
            full(_K, 2 * D_MODEL),
            full(2 * D_MODEL, 2 * D_MODEL),
            full(1, 2 * D_MODEL),
            full(1, 1),
        ],
        out_specs=pl.BlockSpec((t2, 2 * D_MODEL), lambda i: (i, 0)),
        out_shape=jax.ShapeDtypeStruct((n2, 2 * D_MODEL), f32),
    )(mid2, sid2, dur2, pe, po, rhs, vw, beta2, epsc)

    return out.reshape(B, L, D_MODEL)


# speedup vs baseline: 5.2611x; 5.0862x over previous
import jax
import jax.numpy as jnp
from jax.experimental import pallas as pl

D_MODEL = 64
_KM = 32
_KH = 40
_LP = 2
_K = 48


def _encoder_block(mid_ref, sid_ref, dur_ref, rhs_ref, wd_ref,
                   beta_ref, out_ref):
    bf16 = jnp.bfloat16
    f32 = jnp.float32

    b = mid_ref.shape[2]
    rows = jax.lax.broadcasted_iota(jnp.int32, (_KH, b), 0)
    dn = (((1,), (0,)), ((), ()))
    for p in range(_LP):
        m = mid_ref[p]
        s = sid_ref[p]
        d = dur_ref[p]
        oh = ((rows == m) | (rows == s + _KM)).astype(bf16)
        lhs = jnp.concatenate(
            [d.astype(bf16), jnp.zeros((7, b), bf16), oh], axis=0)
        c = jax.lax.dot_general(rhs_ref[...], lhs, dn,
                                preferred_element_type=f32)
        sq = c * c * wd_ref[...]
        var = jnp.sum(sq, axis=0, keepdims=True) + 1e-5
        out_ref[p] = c * jax.lax.rsqrt(var) + beta_ref[...]


def kernel(durations, machine_ids, statuses, W_dur, b_dur,
           machine_table, status_table, gamma, beta):
    B, L, _ = durations.shape
    f32 = jnp.float32
    bf16 = jnp.bfloat16

    midT = machine_ids.astype(jnp.int32).T.reshape(L, 1, B)
    sidT = statuses.astype(jnp.int32).T.reshape(L, 1, B)
    durT = durations.reshape(B, L).T.reshape(L, 1, B)

    cmat = (jnp.eye(D_MODEL, dtype=f32)
            - jnp.full((D_MODEL, D_MODEL), 1.0 / D_MODEL, f32)) * gamma
    mtab = jnp.matmul(machine_table, cmat)
    stab = jnp.matmul(status_table + b_dur, cmat)
    wc = jnp.matmul(W_dur, cmat)
    tab = jnp.zeros((_K, D_MODEL), f32)
    tab = tab.at[0:1].set(wc)
    tab = tab.at[8:8 + mtab.shape[0]].set(mtab)
    tab = tab.at[8 + _KM:8 + _KM + stab.shape[0]].set(stab)
    rhs = tab.T.astype(bf16)

    wd = (1.0 / (D_MODEL * gamma * gamma)).reshape(D_MODEL, 1)
    betac = beta.reshape(D_MODEL, 1)

    blk3 = lambda i: (i, 0, 0)
    full = lambda *shape: pl.BlockSpec(shape, lambda i: (0,) * len(shape))

    out = pl.pallas_call(
        _encoder_block,
        grid=(L // _LP,),
        in_specs=[pl.BlockSpec((_LP, 1, B), blk3)] * 3 + [
            full(D_MODEL, _K),
            full(D_MODEL, 1),
            full(D_MODEL, 1),
        ],
        out_specs=pl.BlockSpec((_LP, D_MODEL, B), blk3),
        out_shape=jax.ShapeDtypeStruct((L, D_MODEL, B), f32),
    )(midT, sidT, durT, rhs, wd, betac)

    return jnp.transpose(out, (2, 0, 1))


# speedup vs baseline: 6.2967x; 1.1968x over previous
import jax
import jax.numpy as jnp
from jax.experimental import pallas as pl

D_MODEL = 64
_KM = 32
_KH = 40
_LP = 4
_K = 48


def _encoder_block(mid_ref, sid_ref, dur_ref, rhs_ref, wd_ref,
                   beta_ref, out_ref):
    bf16 = jnp.bfloat16
    f32 = jnp.float32

    b = mid_ref.shape[2]
    rows = jax.lax.broadcasted_iota(jnp.int32, (_KH, b), 0)
    dn = (((1,), (0,)), ((), ()))
    for p in range(_LP):
        m = mid_ref[p]
        s = sid_ref[p]
        d = dur_ref[p]
        oh = ((rows == m) | (rows == s + _KM)).astype(bf16)
        lhs = jnp.concatenate(
            [d.astype(bf16), jnp.zeros((7, b), bf16), oh], axis=0)
        c = jax.lax.dot_general(rhs_ref[...], lhs, dn,
                                preferred_element_type=f32)
        sq = c * c * wd_ref[...]
        var = jnp.sum(sq, axis=0, keepdims=True) + 1e-5
        out_ref[p] = c * jax.lax.rsqrt(var) + beta_ref[...]


def kernel(durations, machine_ids, statuses, W_dur, b_dur,
           machine_table, status_table, gamma, beta):
    B, L, _ = durations.shape
    f32 = jnp.float32
    bf16 = jnp.bfloat16

    midT = machine_ids.astype(jnp.int32).T.reshape(L, 1, B)
    sidT = statuses.astype(jnp.int32).T.reshape(L, 1, B)
    durT = durations.reshape(B, L).T.reshape(L, 1, B)

    cmat = (jnp.eye(D_MODEL, dtype=f32)
            - jnp.full((D_MODEL, D_MODEL), 1.0 / D_MODEL, f32)) * gamma
    mtab = jnp.matmul(machine_table, cmat)
    stab = jnp.matmul(status_table + b_dur, cmat)
    wc = jnp.matmul(W_dur, cmat)
    tab = jnp.zeros((_K, D_MODEL), f32)
    tab = tab.at[0:1].set(wc)
    tab = tab.at[8:8 + mtab.shape[0]].set(mtab)
    tab = tab.at[8 + _KM:8 + _KM + stab.shape[0]].set(stab)
    rhs = tab.T.astype(bf16)

    wd = (1.0 / (D_MODEL * gamma * gamma)).reshape(D_MODEL, 1)
    betac = beta.reshape(D_MODEL, 1)

    blk3 = lambda i: (i, 0, 0)
    full = lambda *shape: pl.BlockSpec(shape, lambda i: (0,) * len(shape))

    out = pl.pallas_call(
        _encoder_block,
        grid=(L // _LP,),
        in_specs=[pl.BlockSpec((_LP, 1, B), blk3)] * 3 + [
            full(D_MODEL, _K),
            full(D_MODEL, 1),
            full(D_MODEL, 1),
        ],
        out_specs=pl.BlockSpec((_LP, D_MODEL, B), blk3),
        out_shape=jax.ShapeDtypeStruct((L, D_MODEL, B), f32),
    )(midT, sidT, durT, rhs, wd, betac)

    return jnp.transpose(out, (2, 0, 1))
